# Initial kernel scaffold; baseline (speedup 1.0000x reference)
import jax
import jax.numpy as jnp
from jax.experimental import pallas as pl


def kernel(loc_data, conf_data, priors):
    raise NotImplementedError("write your pallas kernel here")


# speedup vs baseline: 26.2182x; 26.2182x over previous
import jax
import jax.numpy as jnp
from jax.experimental import pallas as pl
from jax.experimental.pallas import tpu as pltpu

_CONF = 0.01
_NMS = 0.45
_V0 = 0.1
_V1 = 0.2
_K = 200
_NEG = -1e30


def _nms_kernel(locT_ref, scores_ref, priorsT_ref, out_ref,
                bx0_ref, by0_ref, bx1_ref, by1_ref, area_ref, m_ref):
    p0 = priorsT_ref[0][None, :]
    p1 = priorsT_ref[1][None, :]
    p2 = priorsT_ref[2][None, :]
    p3 = priorsT_ref[3][None, :]
    l0 = locT_ref[0]
    l1 = locT_ref[1]
    l2 = locT_ref[2]
    l3 = locT_ref[3]
    cx = p0 + l0 * _V0 * p2
    cy = p1 + l1 * _V0 * p3
    w = p2 * jnp.exp(l2 * _V1)
    h = p3 * jnp.exp(l3 * _V1)
    x0 = cx - w / 2.0
    y0 = cy - h / 2.0
    x1 = x0 + w
    y1 = y0 + h
    bx0_ref[...] = x0
    by0_ref[...] = y0
    bx1_ref[...] = x1
    by1_ref[...] = y1
    area_ref[...] = jnp.maximum(x1 - x0, 0.0) * jnp.maximum(y1 - y0, 0.0)
    s = scores_ref[...]
    m_ref[...] = jnp.where(s > _CONF, s, _NEG)

    B, N = s.shape
    lane = jax.lax.broadcasted_iota(jnp.int32, (B, N), 1)

    def step(t, carry):
        m = m_ref[...]
        sel = jnp.max(m, axis=1, keepdims=True)
        idx = jnp.min(jnp.where(m == sel, lane, N), axis=1, keepdims=True)
        onehot = lane == idx
        ohf = jnp.where(onehot, 1.0, 0.0)
        bx0 = bx0_ref[...]
        by0 = by0_ref[...]
        bx1 = bx1_ref[...]
        by1 = by1_ref[...]
        ar = area_ref[...]
        sx0 = jnp.sum(ohf * bx0, axis=1, keepdims=True)
        sy0 = jnp.sum(ohf * by0, axis=1, keepdims=True)
        sx1 = jnp.sum(ohf * bx1, axis=1, keepdims=True)
        sy1 = jnp.sum(ohf * by1, axis=1, keepdims=True)
        sar = jnp.maximum(sx1 - sx0, 0.0) * jnp.maximum(sy1 - sy0, 0.0)
        ltx = jnp.maximum(sx0, bx0)
        lty = jnp.maximum(sy0, by0)
        rbx = jnp.minimum(sx1, bx1)
        rby = jnp.minimum(sy1, by1)
        iw = jnp.maximum(rbx - ltx, 0.0)
        ih = jnp.maximum(rby - lty, 0.0)
        inter = iw * ih
        iou = inter / (sar + ar - inter + 1e-12)
        valid = sel > _CONF
        new_m = jnp.where(iou > _NMS, _NEG, m)
        new_m = jnp.where(onehot, _NEG, new_m)
        m_ref[...] = jnp.where(valid, new_m, m)
        row = jnp.concatenate([sel, sx0, sy0, sx1, sy1], axis=1)
        row = jnp.where(valid, row, 0.0)
        out_ref[pl.ds(t, 1)] = row[None]
        return carry

    jax.lax.fori_loop(0, _K, step, 0)


def kernel(loc_data, conf_data, priors):
    B, N, _ = loc_data.shape
    locT = jnp.transpose(loc_data, (2, 0, 1))
    scores = conf_data[:, :, 1]
    priorsT = jnp.transpose(priors, (1, 0))

    out_k = pl.pallas_call(
        _nms_kernel,
        out_shape=jax.ShapeDtypeStruct((_K, B, 5), jnp.float32),
        scratch_shapes=[pltpu.VMEM((B, N), jnp.float32)] * 6,
    )(locT, scores, priorsT)

    cls1 = jnp.transpose(out_k, (1, 0, 2))
    cls0 = jnp.zeros_like(cls1)
    return jnp.stack([cls0, cls1], axis=1)


# speedup vs baseline: 74.6411x; 2.8469x over previous
import functools

import jax
import jax.numpy as jnp
from jax import lax
from jax.experimental import pallas as pl
from jax.experimental.pallas import tpu as pltpu
from jax.experimental.pallas import tpu_sc as plsc

_CONF = 0.01
_NMS = 0.45
_V0 = 0.1
_V1 = 0.2
_K = 200
_NEG = -1e30

_NP = 20480
_NBLK = _NP // 256
_SELW = 208
_OUTW = 256


def _decode_kernel(locT_ref, scores_ref, priorsT_ref, m_ref, box_ref, l1_ref):
    p0 = priorsT_ref[0][None, :]
    p1 = priorsT_ref[1][None, :]
    p2 = priorsT_ref[2][None, :]
    p3 = priorsT_ref[3][None, :]
    l0 = locT_ref[0]
    l1 = locT_ref[1]
    l2 = locT_ref[2]
    l3 = locT_ref[3]
    cx = p0 + l0 * _V0 * p2
    cy = p1 + l1 * _V0 * p3
    w = p2 * jnp.exp(l2 * _V1)
    h = p3 * jnp.exp(l3 * _V1)
    x0 = cx - w / 2.0
    y0 = cy - h / 2.0
    x1 = x0 + w
    y1 = y0 + h
    B, N = x0.shape
    box_ref[:, 0, :N] = x0
    box_ref[:, 1, :N] = y0
    box_ref[:, 2, :N] = x1
    box_ref[:, 3, :N] = y1
    box_ref[:, :, N:] = jnp.zeros((B, 4, _NP - N), jnp.float32)
    s = scores_ref[...]
    m_ref[:, :N] = jnp.where(s > _CONF, s, _NEG)
    m_ref[:, N:] = jnp.full((B, _NP - N), _NEG, jnp.float32)
    mfull = m_ref[...]
    for i in range(_NBLK):
        blkmax = jnp.max(mfull[:, i * 256:(i + 1) * 256], axis=1, keepdims=True)
        l1_ref[:, i:i + 1] = blkmax


def _sc_nms_kernel(m_hbm, box_hbm, l1_hbm, out_hbm, m_v, box_v, l1_v, sel_v, out_v):
    B = m_hbm.shape[0]
    wid = lax.axis_index("s") * 2 + lax.axis_index("c")

    @pl.when(wid < B)
    def _():
        b = wid
        pltpu.sync_copy(m_hbm.at[b], m_v)
        pltpu.sync_copy(box_hbm.at[b], box_v)
        pltpu.sync_copy(l1_hbm.at[b], l1_v)

        z16 = jnp.zeros((16,), jnp.float32)
        for i in range(5 * _OUTW // 16):
            out_v[pl.ds(i * 16, 16)] = z16
        for i in range(5 * _SELW // 16):
            sel_v[pl.ds(i * 16, 16)] = z16

        iota16 = lax.iota(jnp.int32, 16)

        def cond(carry):
            count, going = carry
            return (count < _K) & going

        def body(carry):
            count, _ = carry
            v0 = l1_v[pl.ds(0, 16)]
            v1 = l1_v[pl.ds(16, 16)]
            v2 = l1_v[pl.ds(32, 16)]
            v3 = l1_v[pl.ds(48, 16)]
            v4 = l1_v[pl.ds(64, 16)]
            mm = jnp.maximum(jnp.maximum(jnp.maximum(v0, v1),
                                         jnp.maximum(v2, v3)), v4)
            gmax = jnp.max(mm)
            going = gmax > _CONF
            gvec = jnp.full((16,), gmax)

            blk = jnp.int32(9999)
            for k, vk in enumerate((v0, v1, v2, v3, v4)):
                eq = vk == gvec
                f = jnp.min(jnp.where(eq, iota16, 16))
                blk = jnp.minimum(blk, jnp.where(f < 16, k * 16 + f, 9999))

            base = blk * 256
            off = jnp.int32(9999)
            for j in range(16):
                row = m_v[pl.ds(base + j * 16, 16)]
                eqr = row == gvec
                fj = jnp.min(jnp.where(eqr, iota16, 16))
                off = jnp.minimum(off, jnp.where(fj < 16, j * 16 + fj, 9999))
            idx = base + off

            idxv = jnp.full((16,), idx)
            cx0v = plsc.load_gather(box_v, [jnp.zeros((16,), jnp.int32), idxv])
            cy0v = plsc.load_gather(box_v, [jnp.full((16,), 1, jnp.int32), idxv])
            cx1v = plsc.load_gather(box_v, [jnp.full((16,), 2, jnp.int32), idxv])
            cy1v = plsc.load_gather(box_v, [jnp.full((16,), 3, jnp.int32), idxv])
            cav = jnp.maximum(cx1v - cx0v, 0.0) * jnp.maximum(cy1v - cy0v, 0.0)

            nrows = lax.shift_right_logical(count + 15, 4)

            def iou_body(j, rej):
                sx0 = sel_v[pl.ds(0 * _SELW + j * 16, 16)]
                sy0 = sel_v[pl.ds(1 * _SELW + j * 16, 16)]
                sx1 = sel_v[pl.ds(2 * _SELW + j * 16, 16)]
                sy1 = sel_v[pl.ds(3 * _SELW + j * 16, 16)]
                sar = sel_v[pl.ds(4 * _SELW + j * 16, 16)]
                iw = jnp.maximum(jnp.minimum(cx1v, sx1) - jnp.maximum(cx0v, sx0), 0.0)
                ih = jnp.maximum(jnp.minimum(cy1v, sy1) - jnp.maximum(cy0v, sy0), 0.0)
                inter = iw * ih
                iou = inter / (cav + sar - inter + 1e-12)
                return rej | (jnp.max(iou) > _NMS)

            rejected = lax.fori_loop(0, nrows, iou_body, False)
            do_acc = going & jnp.logical_not(rejected)

            @pl.when(do_acc)
            def _():
                selval = jnp.where(iota16 == 0, cx0v,
                         jnp.where(iota16 == 1, cy0v,
                         jnp.where(iota16 == 2, cx1v,
                         jnp.where(iota16 == 3, cy1v, cav))))
                outval = jnp.where(iota16 == 0, gvec,
                         jnp.where(iota16 == 1, cx0v,
                         jnp.where(iota16 == 2, cy0v,
                         jnp.where(iota16 == 3, cx1v, cy1v))))
                lane5 = iota16 < 5
                lidx = jnp.where(lane5, iota16, 0)
                plsc.store_scatter(sel_v, [lidx * _SELW + count], selval, mask=lane5)
                plsc.store_scatter(out_v, [lidx * _OUTW + count], outval, mask=lane5)

            @pl.when(going)
            def _():
                lane0 = iota16 == 0
                plsc.store_scatter(m_v, [idxv], jnp.full((16,), _NEG, jnp.float32),
                                   mask=lane0)
                t = m_v[pl.ds(base, 16)]
                for j in range(1, 16):
                    t = jnp.maximum(t, m_v[pl.ds(base + j * 16, 16)])
                plsc.store_scatter(l1_v, [jnp.full((16,), blk)],
                                   jnp.full((16,), jnp.max(t)), mask=lane0)

            return (jnp.where(do_acc, count + 1, count), going)

        lax.while_loop(cond, body, (jnp.int32(0), jnp.bool_(True)))
        pltpu.sync_copy(out_v, out_hbm.at[b])


def kernel(loc_data, conf_data, priors):
    B, N, _ = loc_data.shape
    locT = jnp.transpose(loc_data, (2, 0, 1))
    scores = conf_data[:, :, 1]
    priorsT = jnp.transpose(priors, (1, 0))

    m, box, l1 = pl.pallas_call(
        _decode_kernel,
        out_shape=[
            jax.ShapeDtypeStruct((B, _NP), jnp.float32),
            jax.ShapeDtypeStruct((B, 4, _NP), jnp.float32),
            jax.ShapeDtypeStruct((B, _NBLK), jnp.float32),
        ],
    )(locT, scores, priorsT)

    sc = pl.kernel(
        _sc_nms_kernel,
        out_type=jax.ShapeDtypeStruct((B, 5 * _OUTW), jnp.float32),
        mesh=plsc.VectorSubcoreMesh(core_axis_name="c", subcore_axis_name="s"),
        compiler_params=pltpu.CompilerParams(needs_layout_passes=False),
        scratch_types=[
            pltpu.VMEM((_NP,), jnp.float32),
            pltpu.VMEM((4, _NP), jnp.float32),
            pltpu.VMEM((_NBLK,), jnp.float32),
            pltpu.VMEM((5 * _SELW,), jnp.float32),
            pltpu.VMEM((5 * _OUTW,), jnp.float32),
        ],
    )
    out_flat = sc(m, box, l1)

    fields = out_flat.reshape(B, 5, _OUTW)[:, :, :_K]
    cls1 = jnp.transpose(fields, (0, 2, 1))
    cls0 = jnp.zeros_like(cls1)
    return jnp.stack([cls0, cls1], axis=1)


# speedup vs baseline: 76.0025x; 1.0182x over previous
import jax
import jax.numpy as jnp
from jax import lax
from jax.experimental import pallas as pl
from jax.experimental.pallas import tpu as pltpu
from jax.experimental.pallas import tpu_sc as plsc

_CONF = 0.01
_NMS = 0.45
_V0 = 0.1
_V1 = 0.2
_K = 200
_NEG = -1e30

_NP = 20480
_NB1 = _NP // 16
_NB2 = _NP // 256
_SELW = 208
_OUTW = 256


def _decode_kernel(locT_ref, scores_ref, priorsT_ref,
                   m_ref, box_ref, l1_ref, l2_ref):
    p0 = priorsT_ref[0][None, :]
    p1 = priorsT_ref[1][None, :]
    p2 = priorsT_ref[2][None, :]
    p3 = priorsT_ref[3][None, :]
    l0 = locT_ref[0]
    l1 = locT_ref[1]
    l2 = locT_ref[2]
    l3 = locT_ref[3]
    cx = p0 + l0 * _V0 * p2
    cy = p1 + l1 * _V0 * p3
    w = p2 * jnp.exp(l2 * _V1)
    h = p3 * jnp.exp(l3 * _V1)
    x0 = cx - w / 2.0
    y0 = cy - h / 2.0
    x1 = x0 + w
    y1 = y0 + h
    B, N = x0.shape
    box_ref[:, 0, :N] = x0
    box_ref[:, 1, :N] = y0
    box_ref[:, 2, :N] = x1
    box_ref[:, 3, :N] = y1
    box_ref[:, :, N:] = jnp.zeros((B, 4, _NP - N), jnp.float32)
    s = scores_ref[...]
    m_ref[:, :N] = jnp.where(s > _CONF, s, _NEG)
    m_ref[:, N:] = jnp.full((B, _NP - N), _NEG, jnp.float32)
    mfull = m_ref[...]
    lvl1 = jnp.max(mfull.reshape(B, _NB1, 16), axis=2)
    l1_ref[...] = lvl1
    l2_ref[...] = jnp.max(lvl1.reshape(B, _NB2, 16), axis=2)


def _sc_nms_kernel(m_hbm, box_hbm, l1_hbm, l2_hbm, out_hbm,
                   m_v, box_v, l1_v, l2_v, sel_v, out_v):
    B = m_hbm.shape[0]
    wid = lax.axis_index("s") * 2 + lax.axis_index("c")

    @pl.when(wid < B)
    def _():
        b = wid
        pltpu.sync_copy(m_hbm.at[b], m_v)
        pltpu.sync_copy(box_hbm.at[b], box_v)
        pltpu.sync_copy(l1_hbm.at[b], l1_v)
        pltpu.sync_copy(l2_hbm.at[b], l2_v)

        z16 = jnp.zeros((16,), jnp.float32)
        for i in range(5 * _OUTW // 16):
            out_v[pl.ds(i * 16, 16)] = z16
        for i in range(5 * _SELW // 16):
            sel_v[pl.ds(i * 16, 16)] = z16

        iota16 = lax.iota(jnp.int32, 16)

        def cond(carry):
            count, going = carry
            return (count < _K) & going

        def body(carry):
            count, _ = carry
            v0 = l2_v[pl.ds(0, 16)]
            v1 = l2_v[pl.ds(16, 16)]
            v2 = l2_v[pl.ds(32, 16)]
            v3 = l2_v[pl.ds(48, 16)]
            v4 = l2_v[pl.ds(64, 16)]
            mm = jnp.maximum(jnp.maximum(jnp.maximum(v0, v1),
                                         jnp.maximum(v2, v3)), v4)
            gmax = jnp.max(mm)
            going = gmax > _CONF
            gvec = jnp.full((16,), gmax)

            blk = jnp.int32(9999)
            for k, vk in enumerate((v0, v1, v2, v3, v4)):
                eq = vk == gvec
                f = jnp.min(jnp.where(eq, iota16, 16))
                blk = jnp.minimum(blk, jnp.where(f < 16, k * 16 + f, 9999))

            chunk16 = l1_v[pl.ds(blk * 16, 16)]
            j = jnp.min(jnp.where(chunk16 == gvec, iota16, 16))
            rowstart = blk * 256 + j * 16
            row = m_v[pl.ds(rowstart, 16)]
            lane = jnp.min(jnp.where(row == gvec, iota16, 16))
            idx = rowstart + lane

            idxv = jnp.full((16,), idx)
            cx0v = plsc.load_gather(box_v, [jnp.zeros((16,), jnp.int32), idxv])
            cy0v = plsc.load_gather(box_v, [jnp.full((16,), 1, jnp.int32), idxv])
            cx1v = plsc.load_gather(box_v, [jnp.full((16,), 2, jnp.int32), idxv])
            cy1v = plsc.load_gather(box_v, [jnp.full((16,), 3, jnp.int32), idxv])
            cav = jnp.maximum(cx1v - cx0v, 0.0) * jnp.maximum(cy1v - cy0v, 0.0)

            nrows = lax.shift_right_logical(count + 15, 4)

            def iou_body(jj, acc):
                sx0 = sel_v[pl.ds(0 * _SELW + jj * 16, 16)]
                sy0 = sel_v[pl.ds(1 * _SELW + jj * 16, 16)]
                sx1 = sel_v[pl.ds(2 * _SELW + jj * 16, 16)]
                sy1 = sel_v[pl.ds(3 * _SELW + jj * 16, 16)]
                sar = sel_v[pl.ds(4 * _SELW + jj * 16, 16)]
                iw = jnp.maximum(jnp.minimum(cx1v, sx1) - jnp.maximum(cx0v, sx0), 0.0)
                ih = jnp.maximum(jnp.minimum(cy1v, sy1) - jnp.maximum(cy0v, sy0), 0.0)
                inter = iw * ih
                iou = inter / (cav + sar - inter + 1e-12)
                return jnp.maximum(acc, iou)

            maxiou = lax.fori_loop(0, nrows, iou_body, jnp.full((16,), 0.0))
            rejected = jnp.max(maxiou) > _NMS
            do_acc = going & jnp.logical_not(rejected)

            @pl.when(do_acc)
            def _():
                selval = jnp.where(iota16 == 0, cx0v,
                         jnp.where(iota16 == 1, cy0v,
                         jnp.where(iota16 == 2, cx1v,
                         jnp.where(iota16 == 3, cy1v, cav))))
                outval = jnp.where(iota16 == 0, gvec,
                         jnp.where(iota16 == 1, cx0v,
                         jnp.where(iota16 == 2, cy0v,
                         jnp.where(iota16 == 3, cx1v, cy1v))))
                lane5 = iota16 < 5
                lidx = jnp.where(lane5, iota16, 0)
                plsc.store_scatter(sel_v, [lidx * _SELW + count], selval, mask=lane5)
                plsc.store_scatter(out_v, [lidx * _OUTW + count], outval, mask=lane5)

            @pl.when(going)
            def _():
                lane0 = iota16 == 0
                plsc.store_scatter(m_v, [idxv], jnp.full((16,), _NEG, jnp.float32),
                                   mask=lane0)
                nrow = m_v[pl.ds(rowstart, 16)]
                plsc.store_scatter(l1_v, [jnp.full((16,), blk * 16 + j)],
                                   jnp.full((16,), jnp.max(nrow)), mask=lane0)
                nchunk = l1_v[pl.ds(blk * 16, 16)]
                plsc.store_scatter(l2_v, [jnp.full((16,), blk)],
                                   jnp.full((16,), jnp.max(nchunk)), mask=lane0)

            return (jnp.where(do_acc, count + 1, count), going)

        lax.while_loop(cond, body, (jnp.int32(0), jnp.bool_(True)))
        pltpu.sync_copy(out_v, out_hbm.at[b])


def kernel(loc_data, conf_data, priors):
    B, N, _ = loc_data.shape
    locT = jnp.transpose(loc_data, (2, 0, 1))
    scores = conf_data[:, :, 1]
    priorsT = jnp.transpose(priors, (1, 0))

    m, box, l1, l2 = pl.pallas_call(
        _decode_kernel,
        out_shape=[
            jax.ShapeDtypeStruct((B, _NP), jnp.float32),
            jax.ShapeDtypeStruct((B, 4, _NP), jnp.float32),
            jax.ShapeDtypeStruct((B, _NB1), jnp.float32),
            jax.ShapeDtypeStruct((B, _NB2), jnp.float32),
        ],
    )(locT, scores, priorsT)

    sc = pl.kernel(
        _sc_nms_kernel,
        out_type=jax.ShapeDtypeStruct((B, 5 * _OUTW), jnp.float32),
        mesh=plsc.VectorSubcoreMesh(core_axis_name="c", subcore_axis_name="s"),
        compiler_params=pltpu.CompilerParams(needs_layout_passes=False),
        scratch_types=[
            pltpu.VMEM((_NP,), jnp.float32),
            pltpu.VMEM((4, _NP), jnp.float32),
            pltpu.VMEM((_NB1,), jnp.float32),
            pltpu.VMEM((_NB2,), jnp.float32),
            pltpu.VMEM((5 * _SELW,), jnp.float32),
            pltpu.VMEM((5 * _OUTW,), jnp.float32),
        ],
    )
    out_flat = sc(m, box, l1, l2)

    fields = out_flat.reshape(B, 5, _OUTW)[:, :, :_K]
    cls1 = jnp.transpose(fields, (0, 2, 1))
    cls0 = jnp.zeros_like(cls1)
    return jnp.stack([cls0, cls1], axis=1)
